# baseline (speedup 1.0000x reference)
import functools

import jax
import jax.numpy as jnp
from jax import lax
from jax.experimental import pallas as pl
from jax.experimental.pallas import tpu as pltpu
from jax.experimental.pallas import tpu_sc as plsc

BATCH = 16384
LATENT_DIM = 16
NUM_CORES = 2
NUM_SUBCORES = 16
NUM_WORKERS = NUM_CORES * NUM_SUBCORES
ROWS_PER_WORKER = BATCH // NUM_WORKERS
IDX_CHUNK = 128
NUM_CHUNKS = ROWS_PER_WORKER // IDX_CHUNK
GROUPS = ROWS_PER_WORKER // LATENT_DIM


def _gmf_body(uidx_hbm, iidx_hbm, emb_u_hbm, emb_i_hbm, wrot_hbm, b_hbm,
              out_hbm, uidx_v, iidx_v, u_rows, i_rows, wrot_v, b_v, out_v, sem):
    wid = lax.axis_index("c") * NUM_SUBCORES + lax.axis_index("s")

    pltpu.sync_copy(uidx_hbm.at[wid], uidx_v)
    pltpu.sync_copy(iidx_hbm.at[wid], iidx_v)
    pltpu.sync_copy(wrot_hbm, wrot_v)
    pltpu.sync_copy(b_hbm, b_v)

    descs = []
    for j in range(NUM_CHUNKS):
        dst = pl.ds(j * IDX_CHUNK, IDX_CHUNK)
        descs.append(pltpu.async_copy(emb_u_hbm.at[uidx_v.at[j]], u_rows.at[dst], sem))
        descs.append(pltpu.async_copy(emb_i_hbm.at[iidx_v.at[j]], i_rows.at[dst], sem))
    for d in descs:
        d.wait()

    iota16 = lax.iota(jnp.int32, 16)
    col_ids = [(iota16 + d) & 15 for d in range(LATENT_DIM)]
    w_rots = [wrot_v[d] for d in range(LATENT_DIM)]
    b_reg = b_v[...]

    def group(g, carry):
        row_ids = g * 16 + iota16
        acc = jnp.zeros((16,), jnp.float32)
        for d in range(LATENT_DIM):
            uc = plsc.load_gather(u_rows, [row_ids, col_ids[d]])
            ic = plsc.load_gather(i_rows, [row_ids, col_ids[d]])
            acc = acc + uc * ic * w_rots[d]
        logits = acc + b_reg
        rating = 1.0 / (1.0 + jnp.exp(-logits))
        out_v[pl.ds(g * 16, 16)] = rating
        return carry

    lax.fori_loop(0, GROUPS, group, 0)

    pltpu.sync_copy(out_v, out_hbm.at[pl.ds(wid * ROWS_PER_WORKER, ROWS_PER_WORKER)])


@jax.jit
def _gmf(uidx3, iidx3, emb_u, emb_i, wrot, b16):
    mesh = plsc.VectorSubcoreMesh(core_axis_name="c", subcore_axis_name="s")
    f = functools.partial(
        pl.kernel,
        mesh=mesh,
        out_type=jax.ShapeDtypeStruct((BATCH,), jnp.float32),
        compiler_params=pltpu.CompilerParams(
            needs_layout_passes=False, use_tc_tiling_on_sc=False),
        scratch_types=[
            pltpu.VMEM((NUM_CHUNKS, IDX_CHUNK), jnp.int32),
            pltpu.VMEM((NUM_CHUNKS, IDX_CHUNK), jnp.int32),
            pltpu.VMEM((ROWS_PER_WORKER, LATENT_DIM), jnp.float32),
            pltpu.VMEM((ROWS_PER_WORKER, LATENT_DIM), jnp.float32),
            pltpu.VMEM((LATENT_DIM, LATENT_DIM), jnp.float32),
            pltpu.VMEM((LATENT_DIM,), jnp.float32),
            pltpu.VMEM((ROWS_PER_WORKER,), jnp.float32),
            pltpu.SemaphoreType.DMA,
        ],
    )(_gmf_body)
    return f(uidx3, iidx3, emb_u, emb_i, wrot, b16)


def kernel(user_indices, item_indices, domain_idc, embedding_user,
           embedding_item, affine_W, affine_b):
    del domain_idc
    uidx3 = user_indices.reshape(NUM_WORKERS, NUM_CHUNKS, IDX_CHUNK)
    iidx3 = item_indices.reshape(NUM_WORKERS, NUM_CHUNKS, IDX_CHUNK)
    w16 = affine_W.reshape(LATENT_DIM)
    wrot = jnp.stack([jnp.roll(w16, -d) for d in range(LATENT_DIM)])
    b16 = jnp.broadcast_to(affine_b, (LATENT_DIM,))
    out = _gmf(uidx3, iidx3, embedding_user, embedding_item, wrot, b16)
    return out.reshape(BATCH, 1)
